# Initial kernel scaffold; baseline (speedup 1.0000x reference)
import jax
import jax.numpy as jnp
from jax.experimental import pallas as pl


def kernel(x, tok_table, pos_table):
    raise NotImplementedError("write your pallas kernel here")


# speedup vs baseline: 1.9069x; 1.9069x over previous
import functools

import jax
import jax.numpy as jnp
from jax import lax
from jax.experimental import pallas as pl
from jax.experimental.pallas import tpu as pltpu
from jax.experimental.pallas import tpu_sc as plsc

NROWS = 4096
ROWLEN = 200
DIM = 64
NTOK = NROWS * ROWLEN

NC = 2
NS = 16
NW = NC * NS
ROWS_PER_W = NROWS // NW
BLK_ROWS = 2
T = BLK_ROWS * ROWLEN
NBLK = ROWS_PER_W // BLK_ROWS
TOK_PER_W = ROWS_PER_W * ROWLEN

_mesh = plsc.VectorSubcoreMesh(
    core_axis_name="c", subcore_axis_name="s", num_cores=NC, num_subcores=NS)


@functools.partial(
    pl.kernel,
    out_type=jax.ShapeDtypeStruct((NTOK, DIM), jnp.float32),
    mesh=_mesh,
    compiler_params=pltpu.CompilerParams(
        use_tc_tiling_on_sc=False, needs_layout_passes=False),
    scratch_types=[
        pltpu.VMEM((T,), jnp.int32),
        pltpu.VMEM((T, DIM), jnp.float32),
        pltpu.VMEM((T, DIM), jnp.float32),
        pltpu.SemaphoreType.DMA,
        pltpu.SemaphoreType.DMA,
    ],
)
def _emb_kernel(x_hbm, tok_hbm, pos_hbm, out_hbm,
                idx_v, tok_rows, pos_rows, gsem, psem):
    wid = lax.axis_index("s") * NC + lax.axis_index("c")
    zero = jnp.zeros((16,), jnp.int32)
    lane = lax.iota(jnp.int32, 16)

    def block(g, carry_unused):
        tok0 = wid * TOK_PER_W + g * T
        pltpu.sync_copy(x_hbm.at[pl.ds(tok0, T)], idx_v)

        cps = []
        for r in range(BLK_ROWS):
            base = r * ROWLEN
            carry = zero
            for c in range(13):
                o = base + (c * 16 if c < 12 else ROWLEN - 16)
                xi = idx_v[pl.ds(o, 16)]
                mi = jnp.minimum(xi, 1)
                cs = plsc.cumsum(mi)
                pos = mi * (cs + carry)
                cps.append(pltpu.async_copy(
                    tok_hbm.at[xi], tok_rows.at[pl.ds(o, 16)], gsem))
                cps.append(pltpu.async_copy(
                    pos_hbm.at[pos], pos_rows.at[pl.ds(o, 16)], psem))
                if c < 11:
                    carry = carry + jnp.broadcast_to(jnp.sum(mi), (16,))
                elif c == 11:
                    head = 1 - lane // 8
                    carry = carry + jnp.broadcast_to(jnp.sum(mi * head), (16,))

        for cp in cps:
            cp.wait()

        def addb(t, c):
            for k in range(DIM // 16):
                s = pl.ds(k * 16, 16)
                tok_rows[t, s] = tok_rows[t, s] + pos_rows[t, s]
            return c
        lax.fori_loop(0, T, addb, 0)

        pltpu.sync_copy(tok_rows, out_hbm.at[pl.ds(tok0, T)])
        return carry_unused

    lax.fori_loop(0, NBLK, block, 0)


def kernel(x, tok_table, pos_table):
    x = x.astype(jnp.int32)
    emb = _emb_kernel(x.reshape(NTOK), tok_table, pos_table)
    return emb.reshape(NROWS, ROWLEN, DIM), x == 0
